# baseline (speedup 1.0000x reference)
import functools

import jax
import jax.numpy as jnp
from jax import lax
from jax.experimental import pallas as pl
from jax.experimental.pallas import tpu as pltpu
from jax.experimental.pallas import tpu_sc as plsc

NUM_TABLES = 4
BUCKETS_C = 1000000
EMBED = 16
B_, T_ = 4096, 50
NPOS = B_ * T_
SALT = 3266489917

_info = plsc.get_sparse_core_info()
NC, NS, L = _info.num_cores, _info.num_subcores, _info.num_lanes
NW = NC * NS
PER_W = NPOS // NW
CHUNK = 640
NCHUNK = PER_W // CHUNK
NLOOK = NUM_TABLES * CHUNK
IDX_ROWS = NLOOK // 128

M15625 = 15625
C2_16 = 2**16 % M15625
C2_32 = 2**32 % M15625
INV15625_64 = 57
SALT_HI = SALT >> 16
SALT_LO = SALT & 0xFFFF


def _u(x):
    return x.astype(jnp.uint32)


def _combo_vecs(s0, s1, s2, s3):
    m = jnp.int32(BUCKETS_C)
    c0 = lax.rem(s0 ^ s1 ^ s2 ^ s3, m)
    c1 = lax.rem(s0 ^ s2, m)
    c2 = lax.rem(s1 ^ s3, m)
    p0 = _u(s0 ^ s1)
    p1 = _u(s2 ^ s3)
    lo = p1 * jnp.uint32(SALT)
    a = p1 >> jnp.uint32(16)
    b = p1 & jnp.uint32(0xFFFF)
    t0 = b * jnp.uint32(SALT_LO)
    t1 = a * jnp.uint32(SALT_LO) + (t0 >> jnp.uint32(16))
    t2 = b * jnp.uint32(SALT_HI) + (t1 & jnp.uint32(0xFFFF))
    hi = a * jnp.uint32(SALT_HI) + (t1 >> jnp.uint32(16)) + (t2 >> jnp.uint32(16))
    c_lo = lo ^ p0
    ch = c_lo >> jnp.uint32(16)
    cl = c_lo & jnp.uint32(0xFFFF)
    r_clo = lax.rem(ch * jnp.uint32(C2_16) + cl, jnp.uint32(M15625))
    r_hi = lax.rem(lax.rem(hi, jnp.uint32(M15625)) * jnp.uint32(C2_32),
                   jnp.uint32(M15625))
    r15625 = lax.rem(r_hi + r_clo, jnp.uint32(M15625))
    r64 = c_lo & jnp.uint32(63)
    k = (((r64 - r15625) & jnp.uint32(63)) * jnp.uint32(INV15625_64)) & jnp.uint32(63)
    c3 = (r15625 + jnp.uint32(M15625) * k).astype(jnp.int32)
    return c0, c1, c2, c3


@functools.partial(
    pl.kernel,
    out_type=jax.ShapeDtypeStruct((NPOS * NUM_TABLES, EMBED), jnp.float32),
    mesh=plsc.VectorSubcoreMesh(core_axis_name="c", subcore_axis_name="s"),
    scratch_types=[
        pltpu.VMEM((NUM_TABLES, CHUNK), jnp.int32),
        pltpu.VMEM((NLOOK,), jnp.int32),
        pltpu.VMEM((NLOOK, EMBED), jnp.float32),
        pltpu.SemaphoreType.DMA,
    ],
    compiler_params=pltpu.CompilerParams(
        needs_layout_passes=False, use_tc_tiling_on_sc=False
    ),
)
def _sc_lookup(sk_hbm, table_hbm, out_hbm, skv, idxv, rows, sem):
    wid = lax.axis_index("s") * jnp.int32(NC) + lax.axis_index("c")
    base_w = wid * jnp.int32(PER_W)
    iota = lax.iota(jnp.int32, L)

    def chunk_body(ci, _):
        base = base_w + ci * jnp.int32(CHUNK)
        pltpu.sync_copy(sk_hbm.at[:, pl.ds(base, CHUNK)], skv)

        def vec_body(i, _):
            off = i * jnp.int32(L)
            s0 = skv[0, pl.ds(off, L)]
            s1 = skv[1, pl.ds(off, L)]
            s2 = skv[2, pl.ds(off, L)]
            s3 = skv[3, pl.ds(off, L)]
            c0, c1, c2, c3 = _combo_vecs(s0, s1, s2, s3)
            fbase = i * jnp.int32(4 * L) + iota * jnp.int32(4)
            for t, c in enumerate((c0, c1, c2, c3)):
                f = fbase + jnp.int32(t)
                gid = c + jnp.int32(t * BUCKETS_C)
                plsc.store_scatter(idxv, [f], gid)
            return jnp.int32(0)

        lax.fori_loop(jnp.int32(0), jnp.int32(CHUNK // L), vec_body, jnp.int32(0))

        copies = [
            pltpu.async_copy(
                table_hbm.at[idxv.at[pl.ds(j * 128, 128)]],
                rows.at[pl.ds(j * 128, 128)],
                sem,
            )
            for j in range(IDX_ROWS)
        ]
        for c in copies:
            c.wait()
        pltpu.sync_copy(rows, out_hbm.at[pl.ds(base * jnp.int32(NUM_TABLES), NLOOK)])
        return jnp.int32(0)

    lax.fori_loop(jnp.int32(0), jnp.int32(NCHUNK), chunk_body, jnp.int32(0))


def kernel(scale_keys, tables):
    sk32 = scale_keys.astype(jnp.int32).reshape(NUM_TABLES, NPOS)
    tbl = tables.reshape(NUM_TABLES * BUCKETS_C, EMBED)
    out = _sc_lookup(sk32, tbl)
    return out.reshape(B_, T_, NUM_TABLES * EMBED)


# speedup vs baseline: 1.5789x; 1.5789x over previous
import functools

import jax
import jax.numpy as jnp
from jax import lax
from jax.experimental import pallas as pl
from jax.experimental.pallas import tpu as pltpu
from jax.experimental.pallas import tpu_sc as plsc

NUM_TABLES = 4
BUCKETS_C = 1000000
EMBED = 16
B_, T_ = 4096, 50
NPOS = B_ * T_
SALT = 3266489917

_info = plsc.get_sparse_core_info()
NC, NS, L = _info.num_cores, _info.num_subcores, _info.num_lanes
NW = NC * NS

PER_W = NPOS // NW
CHUNK = 640
NCHUNK = PER_W // CHUNK
NLOOK = NUM_TABLES * CHUNK
IDX_ROWS = NLOOK // 128

CB = 512
WPT = NW // NUM_TABLES
FULL_PER_J = 124928
NCH_FULL = FULL_PER_J // CB
TAIL0 = 999424
TAIL1 = 999936
CBT = 64

M15625 = 15625
C2_16 = 2**16 % M15625
C2_32 = 2**32 % M15625
INV15625_64 = 57
SALT_HI = SALT >> 16
SALT_LO = SALT & 0xFFFF


def _u(x):
    return x.astype(jnp.uint32)


def _combo_vecs(s0, s1, s2, s3):
    m = jnp.int32(BUCKETS_C)
    c0 = lax.rem(s0 ^ s1 ^ s2 ^ s3, m)
    c1 = lax.rem(s0 ^ s2, m)
    c2 = lax.rem(s1 ^ s3, m)
    p0 = _u(s0 ^ s1)
    p1 = _u(s2 ^ s3)
    lo = p1 * jnp.uint32(SALT)
    a = p1 >> jnp.uint32(16)
    b = p1 & jnp.uint32(0xFFFF)
    t0 = b * jnp.uint32(SALT_LO)
    t1 = a * jnp.uint32(SALT_LO) + (t0 >> jnp.uint32(16))
    t2 = b * jnp.uint32(SALT_HI) + (t1 & jnp.uint32(0xFFFF))
    hi = a * jnp.uint32(SALT_HI) + (t1 >> jnp.uint32(16)) + (t2 >> jnp.uint32(16))
    c_lo = lo ^ p0
    ch = c_lo >> jnp.uint32(16)
    cl = c_lo & jnp.uint32(0xFFFF)
    r_clo = lax.rem(ch * jnp.uint32(C2_16) + cl, jnp.uint32(M15625))
    r_hi = lax.rem(lax.rem(hi, jnp.uint32(M15625)) * jnp.uint32(C2_32),
                   jnp.uint32(M15625))
    r15625 = lax.rem(r_hi + r_clo, jnp.uint32(M15625))
    r64 = c_lo & jnp.uint32(63)
    k = (((r64 - r15625) & jnp.uint32(63)) * jnp.uint32(INV15625_64)) & jnp.uint32(63)
    c3 = (r15625 + jnp.uint32(M15625) * k).astype(jnp.int32)
    return c0, c1, c2, c3


@functools.partial(
    pl.kernel,
    out_type=jax.ShapeDtypeStruct((NUM_TABLES * BUCKETS_C * EMBED,), jnp.float32),
    mesh=plsc.VectorSubcoreMesh(core_axis_name="c", subcore_axis_name="s"),
    scratch_types=[
        pltpu.VMEM((EMBED, CB), jnp.float32),
        pltpu.VMEM((CB * EMBED,), jnp.float32),
        pltpu.VMEM((EMBED, CBT), jnp.float32),
        pltpu.VMEM((CBT * EMBED,), jnp.float32),
    ],
    compiler_params=pltpu.CompilerParams(
        needs_layout_passes=False, use_tc_tiling_on_sc=True
    ),
)
def _sc_relayout(tt_hbm, r_hbm, slab, oblk, slab_t, oblk_t):
    wid = lax.axis_index("s") * jnp.int32(NC) + lax.axis_index("c")
    t_id = wid // jnp.int32(WPT)
    j = wid - t_id * jnp.int32(WPT)
    iota = lax.iota(jnp.int32, L)
    row0 = t_id * jnp.int32(EMBED)
    bucket0 = j * jnp.int32(FULL_PER_J)

    def do_chunk(b0, sl, ob, width):
        pltpu.sync_copy(tt_hbm.at[pl.ds(row0, EMBED), pl.ds(b0, width)], sl)

        def diag(d, _):
            bcol = (d + iota) & jnp.int32(width - 1)
            vec = plsc.load_gather(sl, [iota, bcol])
            plsc.store_scatter(ob, [bcol * jnp.int32(EMBED) + iota], vec)
            return jnp.int32(0)

        lax.fori_loop(jnp.int32(0), jnp.int32(width), diag, jnp.int32(0))
        dst = (t_id * jnp.int32(BUCKETS_C) + b0) * jnp.int32(EMBED)
        pltpu.sync_copy(ob, r_hbm.at[pl.ds(dst, width * EMBED)])

    def chunk_body(ci, _):
        do_chunk(bucket0 + ci * jnp.int32(CB), slab, oblk, CB)
        return jnp.int32(0)

    lax.fori_loop(jnp.int32(0), jnp.int32(NCH_FULL), chunk_body, jnp.int32(0))

    @pl.when(j == jnp.int32(WPT - 1))
    def _tail():
        do_chunk(jnp.int32(TAIL0), slab, oblk, CB)
        do_chunk(jnp.int32(TAIL1), slab_t, oblk_t, CBT)


@functools.partial(
    pl.kernel,
    out_type=jax.ShapeDtypeStruct((NPOS * NUM_TABLES, EMBED), jnp.float32),
    mesh=plsc.VectorSubcoreMesh(core_axis_name="c", subcore_axis_name="s"),
    scratch_types=[
        pltpu.VMEM((NUM_TABLES, CHUNK), jnp.int32),
        pltpu.VMEM((NLOOK,), jnp.int32),
        pltpu.VMEM((NLOOK, EMBED), jnp.float32),
        pltpu.SemaphoreType.DMA,
    ],
    compiler_params=pltpu.CompilerParams(
        needs_layout_passes=False, use_tc_tiling_on_sc=False
    ),
)
def _sc_lookup(sk_hbm, table_hbm, out_hbm, skv, idxv, rows, sem):
    wid = lax.axis_index("s") * jnp.int32(NC) + lax.axis_index("c")
    base_w = wid * jnp.int32(PER_W)
    iota = lax.iota(jnp.int32, L)

    def chunk_body(ci, _):
        base = base_w + ci * jnp.int32(CHUNK)
        pltpu.sync_copy(sk_hbm.at[:, pl.ds(base, CHUNK)], skv)

        def vec_body(i, _):
            off = i * jnp.int32(L)
            s0 = skv[0, pl.ds(off, L)]
            s1 = skv[1, pl.ds(off, L)]
            s2 = skv[2, pl.ds(off, L)]
            s3 = skv[3, pl.ds(off, L)]
            c0, c1, c2, c3 = _combo_vecs(s0, s1, s2, s3)
            fbase = i * jnp.int32(4 * L) + iota * jnp.int32(4)
            for t, c in enumerate((c0, c1, c2, c3)):
                f = fbase + jnp.int32(t)
                gid = c + jnp.int32(t * BUCKETS_C)
                plsc.store_scatter(idxv, [f], gid)
            return jnp.int32(0)

        lax.fori_loop(jnp.int32(0), jnp.int32(CHUNK // L), vec_body, jnp.int32(0))

        copies = [
            pltpu.async_copy(
                table_hbm.at[idxv.at[pl.ds(j * 128, 128)]],
                rows.at[pl.ds(j * 128, 128)],
                sem,
            )
            for j in range(IDX_ROWS)
        ]
        for c in copies:
            c.wait()
        pltpu.sync_copy(rows, out_hbm.at[pl.ds(base * jnp.int32(NUM_TABLES), NLOOK)])
        return jnp.int32(0)

    lax.fori_loop(jnp.int32(0), jnp.int32(NCHUNK), chunk_body, jnp.int32(0))


def kernel(scale_keys, tables):
    sk32 = scale_keys.astype(jnp.int32).reshape(NUM_TABLES, NPOS)
    tt = jnp.transpose(tables, (0, 2, 1)).reshape(NUM_TABLES * EMBED, BUCKETS_C)
    r_flat = _sc_relayout(tt)
    tbl = r_flat.reshape(NUM_TABLES * BUCKETS_C, EMBED)
    out = _sc_lookup(sk32, tbl)
    return out.reshape(B_, T_, NUM_TABLES * EMBED)


# speedup vs baseline: 2.5048x; 1.5864x over previous
import functools

import jax
import jax.numpy as jnp
from jax import lax
from jax.experimental import pallas as pl
from jax.experimental.pallas import tpu as pltpu
from jax.experimental.pallas import tpu_sc as plsc

NUM_TABLES = 4
BUCKETS_C = 1000000
EMBED = 16
B_, T_ = 4096, 50
NPOS = B_ * T_
SALT = 3266489917

_info = plsc.get_sparse_core_info()
NC, NS, L = _info.num_cores, _info.num_subcores, _info.num_lanes
NW = NC * NS

PER_W = NPOS // NW
CHUNK = 640
NCHUNK = PER_W // CHUNK
NLOOK = NUM_TABLES * CHUNK
IDX_ROWS = NLOOK // 128

CB = 512
WPT = NW // NUM_TABLES
FULL_PER_J = 124928
NCH_FULL = FULL_PER_J // CB
TAIL0 = 999424
TAIL1 = 999936
CBT = 64

M15625 = 15625
C2_16 = 2**16 % M15625
C2_32 = 2**32 % M15625
INV15625_64 = 57
SALT_HI = SALT >> 16
SALT_LO = SALT & 0xFFFF


def _u(x):
    return x.astype(jnp.uint32)


def _combo_vecs(s0, s1, s2, s3):
    m = jnp.int32(BUCKETS_C)
    c0 = lax.rem(s0 ^ s1 ^ s2 ^ s3, m)
    c1 = lax.rem(s0 ^ s2, m)
    c2 = lax.rem(s1 ^ s3, m)
    p0 = _u(s0 ^ s1)
    p1 = _u(s2 ^ s3)
    lo = p1 * jnp.uint32(SALT)
    a = p1 >> jnp.uint32(16)
    b = p1 & jnp.uint32(0xFFFF)
    t0 = b * jnp.uint32(SALT_LO)
    t1 = a * jnp.uint32(SALT_LO) + (t0 >> jnp.uint32(16))
    t2 = b * jnp.uint32(SALT_HI) + (t1 & jnp.uint32(0xFFFF))
    hi = a * jnp.uint32(SALT_HI) + (t1 >> jnp.uint32(16)) + (t2 >> jnp.uint32(16))
    c_lo = lo ^ p0
    ch = c_lo >> jnp.uint32(16)
    cl = c_lo & jnp.uint32(0xFFFF)
    r_clo = lax.rem(ch * jnp.uint32(C2_16) + cl, jnp.uint32(M15625))
    r_hi = lax.rem(lax.rem(hi, jnp.uint32(M15625)) * jnp.uint32(C2_32),
                   jnp.uint32(M15625))
    r15625 = lax.rem(r_hi + r_clo, jnp.uint32(M15625))
    r64 = c_lo & jnp.uint32(63)
    k = (((r64 - r15625) & jnp.uint32(63)) * jnp.uint32(INV15625_64)) & jnp.uint32(63)
    c3 = (r15625 + jnp.uint32(M15625) * k).astype(jnp.int32)
    return c0, c1, c2, c3


SLABW = 640


def _transpose_slab(sl, ob, width):
    iota = lax.iota(jnp.int32, L)
    one = jnp.int32(1)
    sixteen = jnp.int32(EMBED)

    def step(bcol, sidx, mask):
        bc = jnp.maximum(bcol, jnp.int32(0))
        vec = plsc.load_gather(sl, [iota, bc])
        plsc.store_scatter(ob, [sidx], vec, mask=mask)
        return bcol + one, sidx + sixteen

    def pro(_, carry):
        bcol, sidx = carry
        return step(bcol, sidx, bcol >= jnp.int32(0))

    bcol = iota - jnp.int32(15)
    sidx = iota * jnp.int32(17) - jnp.int32(240)
    bcol, sidx = lax.fori_loop(jnp.int32(0), jnp.int32(15), pro, (bcol, sidx))

    def main(_, carry):
        bcol, sidx = carry
        for _u in range(EMBED):
            vec = plsc.load_gather(sl, [iota, bcol])
            plsc.store_scatter(ob, [sidx], vec)
            bcol = bcol + one
            sidx = sidx + sixteen
        return bcol, sidx

    nmain = (width - EMBED) // EMBED
    bcol, sidx = lax.fori_loop(jnp.int32(0), jnp.int32(nmain), main, (bcol, sidx))

    def epi(_, carry):
        bcol, sidx = carry
        return step(bcol, sidx, bcol < jnp.int32(width))

    lax.fori_loop(jnp.int32(0), jnp.int32(EMBED), epi, (bcol, sidx))


@functools.partial(
    pl.kernel,
    out_type=jax.ShapeDtypeStruct((NUM_TABLES * BUCKETS_C * EMBED,), jnp.float32),
    mesh=plsc.VectorSubcoreMesh(core_axis_name="c", subcore_axis_name="s"),
    scratch_types=[
        pltpu.VMEM((EMBED, SLABW), jnp.float32),
        pltpu.VMEM((EMBED, SLABW), jnp.float32),
        pltpu.VMEM((CB * EMBED,), jnp.float32),
        pltpu.VMEM((CB * EMBED,), jnp.float32),
        pltpu.VMEM((CBT * EMBED,), jnp.float32),
        pltpu.SemaphoreType.DMA,
        pltpu.SemaphoreType.DMA,
        pltpu.SemaphoreType.DMA,
        pltpu.SemaphoreType.DMA,
    ],
    compiler_params=pltpu.CompilerParams(
        needs_layout_passes=False, use_tc_tiling_on_sc=True
    ),
)
def _sc_relayout(tt_hbm, tail_hbm, r_hbm, slab0, slab1, oblk0, oblk1, oblk_t,
                 si0, si1, so0, so1):
    wid = lax.axis_index("s") * jnp.int32(NC) + lax.axis_index("c")
    t_id = wid // jnp.int32(WPT)
    j = wid - t_id * jnp.int32(WPT)
    row0 = t_id * jnp.int32(EMBED)
    bucket0 = j * jnp.int32(FULL_PER_J)
    rbase = t_id * jnp.int32(BUCKETS_C)

    def in_src(ci):
        return tt_hbm.at[pl.ds(row0, EMBED), pl.ds(bucket0 + ci * jnp.int32(CB), CB)]

    def out_dst(ci):
        d0 = (rbase + bucket0 + ci * jnp.int32(CB)) * jnp.int32(EMBED)
        return r_hbm.at[pl.ds(d0, CB * EMBED)]

    slabs = (slab0, slab1)
    oblks = (oblk0, oblk1)
    sis = (si0, si1)
    sos = (so0, so1)

    def sl_dst(b):
        return slabs[b].at[:, pl.ds(0, CB)]

    pltpu.async_copy(in_src(jnp.int32(0)), sl_dst(0), sis[0])

    def pair(p, _):
        for b in range(2):
            ci = p * jnp.int32(2) + jnp.int32(b)
            pltpu.make_async_copy(in_src(ci), sl_dst(b), sis[b]).wait()

            @pl.when(ci + jnp.int32(1) < jnp.int32(NCH_FULL))
            def _prefetch():
                pltpu.async_copy(in_src(ci + jnp.int32(1)), sl_dst(1 - b), sis[1 - b])

            @pl.when(p > jnp.int32(0))
            def _drain_out():
                pltpu.make_async_copy(oblks[b], out_dst(ci), sos[b]).wait()

            _transpose_slab(slabs[b], oblks[b], CB)
            pltpu.async_copy(oblks[b], out_dst(ci), sos[b])
        return jnp.int32(0)

    lax.fori_loop(jnp.int32(0), jnp.int32(NCH_FULL // 2), pair, jnp.int32(0))
    last = jnp.int32(NCH_FULL - 2)
    pltpu.make_async_copy(oblk0, out_dst(last), so0).wait()
    pltpu.make_async_copy(oblk1, out_dst(last + jnp.int32(1)), so1).wait()

    @pl.when(j == jnp.int32(WPT - 1))
    def _tail():
        pltpu.sync_copy(
            tt_hbm.at[pl.ds(row0, EMBED), pl.ds(jnp.int32(TAIL0), CB)], sl_dst(0)
        )
        _transpose_slab(slab0, oblk0, CB)
        d0 = (rbase + jnp.int32(TAIL0)) * jnp.int32(EMBED)
        pltpu.sync_copy(oblk0, r_hbm.at[pl.ds(d0, CB * EMBED)])

        pltpu.sync_copy(tail_hbm.at[t_id], oblk_t)
        d1 = (rbase + jnp.int32(TAIL1)) * jnp.int32(EMBED)
        pltpu.sync_copy(oblk_t, r_hbm.at[pl.ds(d1, CBT * EMBED)])


@functools.partial(
    pl.kernel,
    out_type=jax.ShapeDtypeStruct((NPOS * NUM_TABLES, EMBED), jnp.float32),
    mesh=plsc.VectorSubcoreMesh(core_axis_name="c", subcore_axis_name="s"),
    scratch_types=[
        pltpu.VMEM((NUM_TABLES, CHUNK), jnp.int32),
        pltpu.VMEM((NLOOK,), jnp.int32),
        pltpu.VMEM((NLOOK, EMBED), jnp.float32),
        pltpu.SemaphoreType.DMA,
    ],
    compiler_params=pltpu.CompilerParams(
        needs_layout_passes=False, use_tc_tiling_on_sc=False
    ),
)
def _sc_lookup(sk_hbm, table_hbm, out_hbm, skv, idxv, rows, sem):
    wid = lax.axis_index("s") * jnp.int32(NC) + lax.axis_index("c")
    base_w = wid * jnp.int32(PER_W)
    iota = lax.iota(jnp.int32, L)

    def chunk_body(ci, _):
        base = base_w + ci * jnp.int32(CHUNK)
        pltpu.sync_copy(sk_hbm.at[:, pl.ds(base, CHUNK)], skv)

        def vec_body(i, _):
            off = i * jnp.int32(L)
            s0 = skv[0, pl.ds(off, L)]
            s1 = skv[1, pl.ds(off, L)]
            s2 = skv[2, pl.ds(off, L)]
            s3 = skv[3, pl.ds(off, L)]
            c0, c1, c2, c3 = _combo_vecs(s0, s1, s2, s3)
            fbase = i * jnp.int32(4 * L) + iota * jnp.int32(4)
            for t, c in enumerate((c0, c1, c2, c3)):
                f = fbase + jnp.int32(t)
                gid = c + jnp.int32(t * BUCKETS_C)
                plsc.store_scatter(idxv, [f], gid)
            return jnp.int32(0)

        lax.fori_loop(jnp.int32(0), jnp.int32(CHUNK // L), vec_body, jnp.int32(0))

        copies = [
            pltpu.async_copy(
                table_hbm.at[idxv.at[pl.ds(j * 128, 128)]],
                rows.at[pl.ds(j * 128, 128)],
                sem,
            )
            for j in range(IDX_ROWS)
        ]
        for c in copies:
            c.wait()
        pltpu.sync_copy(rows, out_hbm.at[pl.ds(base * jnp.int32(NUM_TABLES), NLOOK)])
        return jnp.int32(0)

    lax.fori_loop(jnp.int32(0), jnp.int32(NCHUNK), chunk_body, jnp.int32(0))


def kernel(scale_keys, tables):
    sk32 = scale_keys.astype(jnp.int32).reshape(NUM_TABLES, NPOS)
    tt = jnp.transpose(tables, (0, 2, 1)).reshape(NUM_TABLES * EMBED, BUCKETS_C)
    tail = tables[:, TAIL1:, :].reshape(NUM_TABLES, CBT * EMBED)
    r_flat = _sc_relayout(tt, tail)
    tbl = r_flat.reshape(NUM_TABLES * BUCKETS_C, EMBED)
    out = _sc_lookup(sk32, tbl)
    return out.reshape(B_, T_, NUM_TABLES * EMBED)


# speedup vs baseline: 2.6582x; 1.0613x over previous
import functools

import jax
import jax.numpy as jnp
from jax import lax
from jax.experimental import pallas as pl
from jax.experimental.pallas import tpu as pltpu
from jax.experimental.pallas import tpu_sc as plsc

NUM_TABLES = 4
BUCKETS_C = 1000000
EMBED = 16
B_, T_ = 4096, 50
NPOS = B_ * T_
SALT = 3266489917

_info = plsc.get_sparse_core_info()
NC, NS, L = _info.num_cores, _info.num_subcores, _info.num_lanes
NW = NC * NS

PER_W = NPOS // NW
CHUNK = 640
NCHUNK = PER_W // CHUNK
NLOOK = NUM_TABLES * CHUNK
IDX_ROWS = NLOOK // 128

CB = 512
WPT = NW // NUM_TABLES
FULL_PER_J = 124928
NCH_FULL = FULL_PER_J // CB
TAIL0 = 999424
TAIL1 = 999936
CBT = 64

M15625 = 15625
C2_16 = 2**16 % M15625
C2_32 = 2**32 % M15625
INV15625_64 = 57
SALT_HI = SALT >> 16
SALT_LO = SALT & 0xFFFF


def _u(x):
    return x.astype(jnp.uint32)


def _combo_vecs(s0, s1, s2, s3):
    m = jnp.int32(BUCKETS_C)
    c0 = lax.rem(s0 ^ s1 ^ s2 ^ s3, m)
    c1 = lax.rem(s0 ^ s2, m)
    c2 = lax.rem(s1 ^ s3, m)
    p0 = _u(s0 ^ s1)
    p1 = _u(s2 ^ s3)
    lo = p1 * jnp.uint32(SALT)
    a = p1 >> jnp.uint32(16)
    b = p1 & jnp.uint32(0xFFFF)
    t0 = b * jnp.uint32(SALT_LO)
    t1 = a * jnp.uint32(SALT_LO) + (t0 >> jnp.uint32(16))
    t2 = b * jnp.uint32(SALT_HI) + (t1 & jnp.uint32(0xFFFF))
    hi = a * jnp.uint32(SALT_HI) + (t1 >> jnp.uint32(16)) + (t2 >> jnp.uint32(16))
    c_lo = lo ^ p0
    ch = c_lo >> jnp.uint32(16)
    cl = c_lo & jnp.uint32(0xFFFF)
    r_clo = lax.rem(ch * jnp.uint32(C2_16) + cl, jnp.uint32(M15625))
    r_hi = lax.rem(lax.rem(hi, jnp.uint32(M15625)) * jnp.uint32(C2_32),
                   jnp.uint32(M15625))
    r15625 = lax.rem(r_hi + r_clo, jnp.uint32(M15625))
    r64 = c_lo & jnp.uint32(63)
    k = (((r64 - r15625) & jnp.uint32(63)) * jnp.uint32(INV15625_64)) & jnp.uint32(63)
    c3 = (r15625 + jnp.uint32(M15625) * k).astype(jnp.int32)
    return c0, c1, c2, c3


SLABW = 640


def _transpose_slab(sl, ob, width):
    iota = lax.iota(jnp.int32, L)
    one = jnp.int32(1)
    sixteen = jnp.int32(EMBED)
    svecs = [iota * jnp.int32(17) + jnp.int32(16 * u) for u in range(8)]

    def step(bcol, sidx, mask):
        bc = jnp.maximum(bcol, jnp.int32(0))
        vec = plsc.load_gather(sl, [iota, bc])
        plsc.store_scatter(ob, [sidx], vec, mask=mask)
        return bcol + one, sidx + sixteen

    def pro(_, carry):
        bcol, sidx = carry
        return step(bcol, sidx, bcol >= jnp.int32(0))

    bcol = iota - jnp.int32(15)
    sidx = iota * jnp.int32(17) - jnp.int32(240)
    bcol, sidx = lax.fori_loop(jnp.int32(0), jnp.int32(15), pro, (bcol, sidx))

    def win(J, bcol):
        obw = ob.at[pl.ds(J * jnp.int32(128), 384)]
        for u in range(8):
            vec = plsc.load_gather(sl, [iota, bcol])
            plsc.store_scatter(obw, [svecs[u]], vec)
            bcol = bcol + one
        return bcol

    nwin = (width - EMBED) // 8
    bcol = lax.fori_loop(jnp.int32(0), jnp.int32(nwin), win, bcol)

    def epi(_, carry):
        bcol, sidx = carry
        return step(bcol, sidx, bcol < jnp.int32(width))

    sidx = iota * jnp.int32(17) + jnp.int32(16 * (width - EMBED))
    lax.fori_loop(jnp.int32(0), jnp.int32(EMBED), epi, (bcol, sidx))


@functools.partial(
    pl.kernel,
    out_type=jax.ShapeDtypeStruct((NUM_TABLES * BUCKETS_C * EMBED,), jnp.float32),
    mesh=plsc.VectorSubcoreMesh(core_axis_name="c", subcore_axis_name="s"),
    scratch_types=[
        pltpu.VMEM((EMBED, SLABW), jnp.float32),
        pltpu.VMEM((EMBED, SLABW), jnp.float32),
        pltpu.VMEM((CB * EMBED,), jnp.float32),
        pltpu.VMEM((CB * EMBED,), jnp.float32),
        pltpu.VMEM((CBT * EMBED,), jnp.float32),
        pltpu.SemaphoreType.DMA,
        pltpu.SemaphoreType.DMA,
        pltpu.SemaphoreType.DMA,
        pltpu.SemaphoreType.DMA,
    ],
    compiler_params=pltpu.CompilerParams(
        needs_layout_passes=False, use_tc_tiling_on_sc=True
    ),
)
def _sc_relayout(tt_hbm, tail_hbm, r_hbm, slab0, slab1, oblk0, oblk1, oblk_t,
                 si0, si1, so0, so1):
    wid = lax.axis_index("s") * jnp.int32(NC) + lax.axis_index("c")
    t_id = wid // jnp.int32(WPT)
    j = wid - t_id * jnp.int32(WPT)
    row0 = t_id * jnp.int32(EMBED)
    bucket0 = j * jnp.int32(FULL_PER_J)
    rbase = t_id * jnp.int32(BUCKETS_C)

    def in_src(ci):
        return tt_hbm.at[pl.ds(row0, EMBED), pl.ds(bucket0 + ci * jnp.int32(CB), CB)]

    def out_dst(ci):
        d0 = (rbase + bucket0 + ci * jnp.int32(CB)) * jnp.int32(EMBED)
        return r_hbm.at[pl.ds(d0, CB * EMBED)]

    slabs = (slab0, slab1)
    oblks = (oblk0, oblk1)
    sis = (si0, si1)
    sos = (so0, so1)

    def sl_dst(b):
        return slabs[b].at[:, pl.ds(0, CB)]

    pltpu.async_copy(in_src(jnp.int32(0)), sl_dst(0), sis[0])

    def pair(p, _):
        for b in range(2):
            ci = p * jnp.int32(2) + jnp.int32(b)
            pltpu.make_async_copy(in_src(ci), sl_dst(b), sis[b]).wait()

            @pl.when(ci + jnp.int32(1) < jnp.int32(NCH_FULL))
            def _prefetch():
                pltpu.async_copy(in_src(ci + jnp.int32(1)), sl_dst(1 - b), sis[1 - b])

            @pl.when(p > jnp.int32(0))
            def _drain_out():
                pltpu.make_async_copy(oblks[b], out_dst(ci), sos[b]).wait()

            _transpose_slab(slabs[b], oblks[b], CB)
            pltpu.async_copy(oblks[b], out_dst(ci), sos[b])
        return jnp.int32(0)

    lax.fori_loop(jnp.int32(0), jnp.int32(NCH_FULL // 2), pair, jnp.int32(0))
    last = jnp.int32(NCH_FULL - 2)
    pltpu.make_async_copy(oblk0, out_dst(last), so0).wait()
    pltpu.make_async_copy(oblk1, out_dst(last + jnp.int32(1)), so1).wait()

    @pl.when(j == jnp.int32(WPT - 1))
    def _tail():
        pltpu.sync_copy(
            tt_hbm.at[pl.ds(row0, EMBED), pl.ds(jnp.int32(TAIL0), CB)], sl_dst(0)
        )
        _transpose_slab(slab0, oblk0, CB)
        d0 = (rbase + jnp.int32(TAIL0)) * jnp.int32(EMBED)
        pltpu.sync_copy(oblk0, r_hbm.at[pl.ds(d0, CB * EMBED)])

        pltpu.sync_copy(tail_hbm.at[t_id], oblk_t)
        d1 = (rbase + jnp.int32(TAIL1)) * jnp.int32(EMBED)
        pltpu.sync_copy(oblk_t, r_hbm.at[pl.ds(d1, CBT * EMBED)])


@functools.partial(
    pl.kernel,
    out_type=jax.ShapeDtypeStruct((NPOS * NUM_TABLES, EMBED), jnp.float32),
    mesh=plsc.VectorSubcoreMesh(core_axis_name="c", subcore_axis_name="s"),
    scratch_types=[
        pltpu.VMEM((NUM_TABLES, CHUNK), jnp.int32),
        pltpu.VMEM((NUM_TABLES, CHUNK), jnp.int32),
        pltpu.VMEM((NLOOK,), jnp.int32),
        pltpu.VMEM((NLOOK,), jnp.int32),
        pltpu.VMEM((NLOOK, EMBED), jnp.float32),
        pltpu.VMEM((NLOOK, EMBED), jnp.float32),
        pltpu.SemaphoreType.DMA,
        pltpu.SemaphoreType.DMA,
        pltpu.SemaphoreType.DMA,
        pltpu.SemaphoreType.DMA,
        pltpu.SemaphoreType.DMA,
        pltpu.SemaphoreType.DMA,
    ],
    compiler_params=pltpu.CompilerParams(
        needs_layout_passes=False, use_tc_tiling_on_sc=False
    ),
)
def _sc_lookup(sk_hbm, table_hbm, out_hbm, skv0, skv1, idxv0, idxv1,
               rows0, rows1, ss0, ss1, gs0, gs1, os0, os1):
    wid = lax.axis_index("s") * jnp.int32(NC) + lax.axis_index("c")
    base_w = wid * jnp.int32(PER_W)
    iota = lax.iota(jnp.int32, L)
    skvs, idxvs, rowss = (skv0, skv1), (idxv0, idxv1), (rows0, rows1)
    sss, gss, oss = (ss0, ss1), (gs0, gs1), (os0, os1)

    def sk_src(ci):
        return sk_hbm.at[:, pl.ds(base_w + ci * jnp.int32(CHUNK), CHUNK)]

    def out_dst(ci):
        d0 = (base_w + ci * jnp.int32(CHUNK)) * jnp.int32(NUM_TABLES)
        return out_hbm.at[pl.ds(d0, NLOOK)]

    def compute(skv, idxv):
        def vec_body(i, _):
            off = i * jnp.int32(L)
            s0 = skv[0, pl.ds(off, L)]
            s1 = skv[1, pl.ds(off, L)]
            s2 = skv[2, pl.ds(off, L)]
            s3 = skv[3, pl.ds(off, L)]
            c0, c1, c2, c3 = _combo_vecs(s0, s1, s2, s3)
            fbase = i * jnp.int32(4 * L) + iota * jnp.int32(4)
            for t, c in enumerate((c0, c1, c2, c3)):
                f = fbase + jnp.int32(t)
                gid = c + jnp.int32(t * BUCKETS_C)
                plsc.store_scatter(idxv, [f], gid)
            return jnp.int32(0)

        lax.fori_loop(jnp.int32(0), jnp.int32(CHUNK // L), vec_body, jnp.int32(0))

    def fire_gathers(b):
        for j in range(IDX_ROWS):
            pltpu.async_copy(
                table_hbm.at[idxvs[b].at[pl.ds(j * 128, 128)]],
                rowss[b].at[pl.ds(j * 128, 128)],
                gss[b],
            )

    def drain_gathers(b):
        pltpu.make_async_copy(table_hbm.at[idxvs[b]], rowss[b], gss[b]).wait()

    pltpu.async_copy(sk_src(jnp.int32(0)), skv0, ss0)

    def pair(p, _):
        for b in range(2):
            ci = p * jnp.int32(2) + jnp.int32(b)
            pltpu.make_async_copy(sk_src(ci), skvs[b], sss[b]).wait()

            @pl.when(ci + jnp.int32(1) < jnp.int32(NCHUNK))
            def _next_sk():
                pltpu.async_copy(sk_src(ci + jnp.int32(1)), skvs[1 - b], sss[1 - b])

            compute(skvs[b], idxvs[b])

            def _after_prev():
                drain_gathers(1 - b)
                pltpu.async_copy(rowss[1 - b], out_dst(ci - jnp.int32(1)), oss[1 - b])

            def _free_rows():
                pltpu.make_async_copy(rowss[b], out_dst(ci), oss[b]).wait()

            if b == 0:
                pl.when(p > jnp.int32(0))(_after_prev)
                pl.when(p > jnp.int32(0))(_free_rows)
            else:
                _after_prev()
                pl.when(p > jnp.int32(0))(_free_rows)
            fire_gathers(b)
        return jnp.int32(0)

    lax.fori_loop(jnp.int32(0), jnp.int32(NCHUNK // 2), pair, jnp.int32(0))
    last = jnp.int32(NCHUNK - 1)
    drain_gathers(1)
    pltpu.async_copy(rows1, out_dst(last), os1)
    pltpu.make_async_copy(rows0, out_dst(last - jnp.int32(1)), os0).wait()
    pltpu.make_async_copy(rows1, out_dst(last), os1).wait()


def kernel(scale_keys, tables):
    sk32 = scale_keys.astype(jnp.int32).reshape(NUM_TABLES, NPOS)
    tt = jnp.transpose(tables, (0, 2, 1)).reshape(NUM_TABLES * EMBED, BUCKETS_C)
    tail = tables[:, TAIL1:, :].reshape(NUM_TABLES, CBT * EMBED)
    r_flat = _sc_relayout(tt, tail)
    tbl = r_flat.reshape(NUM_TABLES * BUCKETS_C, EMBED)
    out = _sc_lookup(sk32, tbl)
    return out.reshape(B_, T_, NUM_TABLES * EMBED)
